# Initial kernel scaffold; baseline (speedup 1.0000x reference)
import jax
import jax.numpy as jnp
from jax.experimental import pallas as pl


def kernel(x, edge_index, W1, b1, W2, b2, Wl, bl):
    raise NotImplementedError("write your pallas kernel here")


# speedup vs baseline: 39.6216x; 39.6216x over previous
import functools

import jax
import jax.numpy as jnp
from jax import lax
from jax.experimental import pallas as pl
from jax.experimental.pallas import tpu as pltpu
from jax.experimental.pallas import tpu_sc as plsc

N = 10000
NPAD = 10240
NC, NS, L = 2, 16, 16
NW = NC * NS
CHUNK = 128
ROWS = 1000

_mesh = plsc.VectorSubcoreMesh(
    core_axis_name="c", subcore_axis_name="s", num_cores=NC, num_subcores=NS)


def _cdiv(a, b):
    return (a + b - 1) // b


def _make_deg(CH):
    @functools.partial(
        pl.kernel,
        out_type=jax.ShapeDtypeStruct((NW, NPAD), jnp.float32),
        mesh=_mesh,
        compiler_params=pltpu.CompilerParams(
            needs_layout_passes=False, use_tc_tiling_on_sc=False),
        scratch_types=[
            pltpu.VMEM((CH, CHUNK), jnp.int32),
            pltpu.VMEM((NPAD,), jnp.float32),
        ],
    )
    def deg_kernel(col_hbm, out_hbm, colv, degv):
        c = lax.axis_index("c")
        s = lax.axis_index("s")
        w = c * NS + s
        pltpu.sync_copy(col_hbm.at[w], colv)

        def zbody(i, carry):
            degv[pl.ds(i * L, L)] = jnp.zeros((L,), jnp.float32)
            return carry

        lax.fori_loop(0, NPAD // L, zbody, 0)
        ones = jnp.ones((L,), jnp.float32)

        def body(j, carry):
            for k in range(CHUNK // L):
                idx = colv[j, pl.ds(k * L, L)]
                plsc.addupdate_scatter(degv, [idx], ones)
            return carry

        lax.fori_loop(0, CH, body, 0)
        pltpu.sync_copy(degv, out_hbm.at[w])

    return deg_kernel


def _make_agg(H, CH):
    rpt = NPAD // NS

    @functools.partial(
        pl.kernel,
        out_type=jax.ShapeDtypeStruct((NC, NPAD, H), jnp.float32),
        mesh=_mesh,
        compiler_params=pltpu.CompilerParams(
            needs_layout_passes=False, use_tc_tiling_on_sc=False),
        scratch_types=[
            pltpu.VMEM((CH, CHUNK), jnp.int32),
            pltpu.VMEM((CH, CHUNK), jnp.int32),
            pltpu.VMEM((CHUNK, H), jnp.float32),
            pltpu.VMEM((CHUNK, H), jnp.float32),
            pltpu.VMEM((CHUNK, H), jnp.float32),
            pltpu.VMEM_SHARED((NPAD, H), jnp.float32),
            pltpu.SemaphoreType.DMA,
            pltpu.SemaphoreType.DMA,
        ],
    )
    def agg_kernel(g_hbm, row_hbm, col_hbm, out_hbm,
                   rowv, colv, buf0, buf1, zbuf, acc, sem0, sem1):
        c = lax.axis_index("c")
        s = lax.axis_index("s")
        w = c * NS + s
        pltpu.sync_copy(row_hbm.at[w], rowv)
        pltpu.sync_copy(col_hbm.at[w], colv)

        def zb(i, carry):
            for k in range(H // L):
                zbuf[i, pl.ds(k * L, L)] = jnp.zeros((L,), jnp.float32)
            return carry

        lax.fori_loop(0, CHUNK, zb, 0)
        for k in range(rpt // CHUNK):
            pltpu.sync_copy(zbuf, acc.at[pl.ds(s * rpt + k * CHUNK, CHUNK)])
        plsc.subcore_barrier()

        pltpu.async_copy(g_hbm.at[rowv.at[0]], buf0, sem0)

        def body(t, carry):
            j = t * 2
            pltpu.async_copy(g_hbm.at[rowv.at[j + 1]], buf1, sem1)
            pltpu.make_async_copy(g_hbm.at[rowv.at[j]], buf0, sem0).wait()
            pltpu.sync_copy(buf0, acc.at[colv.at[j]], add=True)

            @pl.when(j + 2 < CH)
            def _():
                pltpu.async_copy(g_hbm.at[rowv.at[j + 2]], buf0, sem0)

            pltpu.make_async_copy(g_hbm.at[rowv.at[j + 1]], buf1, sem1).wait()
            pltpu.sync_copy(buf1, acc.at[colv.at[j + 1]], add=True)
            return carry

        lax.fori_loop(0, CH // 2, body, 0)
        plsc.subcore_barrier()
        for k in range(rpt // CHUNK):
            r0 = s * rpt + k * CHUNK
            pltpu.sync_copy(acc.at[pl.ds(r0, CHUNK)], zbuf)
            pltpu.sync_copy(zbuf, out_hbm.at[c, pl.ds(r0, CHUNK)])

    return agg_kernel


def _mm_body(x_ref, w_ref, o_ref):
    o_ref[...] = jnp.dot(x_ref[...], w_ref[...],
                         preferred_element_type=jnp.float32)


def _matmul(x, W):
    n, k = x.shape
    m = W.shape[1]
    return pl.pallas_call(
        _mm_body,
        grid=(n // ROWS,),
        in_specs=[pl.BlockSpec((ROWS, k), lambda i: (i, 0)),
                  pl.BlockSpec((k, m), lambda i: (0, 0))],
        out_specs=pl.BlockSpec((ROWS, m), lambda i: (i, 0)),
        out_shape=jax.ShapeDtypeStruct((n, m), jnp.float32),
    )(x, W)


def _scale_body(degp_ref, h_ref, dinv_ref, g_ref):
    deg = jnp.sum(degp_ref[...], axis=1, keepdims=True) + 1.0
    dinv = lax.rsqrt(deg)
    dinv_ref[...] = dinv
    g_ref[...] = h_ref[...] * dinv


def _deg_scale(degT, h1):
    h = h1.shape[1]
    return pl.pallas_call(
        _scale_body,
        grid=(N // ROWS,),
        in_specs=[pl.BlockSpec((ROWS, NW), lambda i: (i, 0)),
                  pl.BlockSpec((ROWS, h), lambda i: (i, 0))],
        out_specs=[pl.BlockSpec((ROWS, 1), lambda i: (i, 0)),
                   pl.BlockSpec((ROWS, h), lambda i: (i, 0))],
        out_shape=[jax.ShapeDtypeStruct((N, 1), jnp.float32),
                   jax.ShapeDtypeStruct((N, h), jnp.float32)],
    )(degT, h1)


def _layer_body(p_ref, g_ref, dinv_ref, b_ref, w_ref, o_ref):
    agg = p_ref[0] + p_ref[1] + g_ref[...]
    outl = jnp.maximum(dinv_ref[...] * agg + b_ref[...], 0.0)
    o_ref[...] = dinv_ref[...] * jnp.dot(outl, w_ref[...],
                                         preferred_element_type=jnp.float32)


def _layer_step(p, g, dinv, b, Wn):
    h = g.shape[1]
    m = Wn.shape[1]
    return pl.pallas_call(
        _layer_body,
        grid=(N // ROWS,),
        in_specs=[pl.BlockSpec((NC, ROWS, h), lambda i: (0, i, 0)),
                  pl.BlockSpec((ROWS, h), lambda i: (i, 0)),
                  pl.BlockSpec((ROWS, 1), lambda i: (i, 0)),
                  pl.BlockSpec((1, h), lambda i: (0, 0)),
                  pl.BlockSpec((h, m), lambda i: (0, 0))],
        out_specs=pl.BlockSpec((ROWS, m), lambda i: (i, 0)),
        out_shape=jax.ShapeDtypeStruct((N, m), jnp.float32),
    )(p, g, dinv, b, Wn)


def _final_body(q_ref, g_ref, dinv_ref, b_ref, wl_ref, bl_ref, o_ref):
    agg = q_ref[0] + q_ref[1] + g_ref[...]
    outl = jnp.maximum(dinv_ref[...] * agg + b_ref[...], 0.0)
    logit = jnp.dot(outl, wl_ref[...],
                    preferred_element_type=jnp.float32) + bl_ref[...]
    o_ref[...] = jnp.concatenate([-logit, logit], axis=1)


def _final_step(q, g, dinv, b, Wl, bl):
    h = g.shape[1]
    return pl.pallas_call(
        _final_body,
        grid=(N // ROWS,),
        in_specs=[pl.BlockSpec((NC, ROWS, h), lambda i: (0, i, 0)),
                  pl.BlockSpec((ROWS, h), lambda i: (i, 0)),
                  pl.BlockSpec((ROWS, 1), lambda i: (i, 0)),
                  pl.BlockSpec((1, h), lambda i: (0, 0)),
                  pl.BlockSpec((h, 1), lambda i: (0, 0)),
                  pl.BlockSpec((1, 1), lambda i: (0, 0))],
        out_specs=pl.BlockSpec((ROWS, 2), lambda i: (i, 0)),
        out_shape=jax.ShapeDtypeStruct((N, 2), jnp.float32),
    )(q, g, dinv, b, Wl, bl)


def kernel(x, edge_index, W1, b1, W2, b2, Wl, bl):
    E = edge_index.shape[1]
    CH = _cdiv(E, NW * CHUNK)
    CH = CH + (CH % 2)
    Epad = NW * CH * CHUNK
    pad = Epad - E
    pad_iota = jnp.arange(pad, dtype=jnp.int32)
    row = jnp.concatenate([edge_index[0], pad_iota % N])
    col = jnp.concatenate([edge_index[1], N + pad_iota % (NPAD - N)])
    row3 = row.reshape(NW, CH, CHUNK)
    col3 = col.reshape(NW, CH, CHUNK)

    degp = _make_deg(CH)(col3)
    h1 = _matmul(x, W1)
    dinv, g1 = _deg_scale(degp[:, :N].T, h1)

    p1 = _make_agg(h1.shape[1], CH)(g1, row3, col3)
    g2 = _layer_step(p1[:, :N, :], g1, dinv, b1.reshape(1, -1), W2)

    p2 = _make_agg(g2.shape[1], CH)(g2, row3, col3)
    return _final_step(p2[:, :N, :], g2, dinv, b2.reshape(1, -1),
                       Wl, bl.reshape(1, 1))


# speedup vs baseline: 45.5055x; 1.1485x over previous
import functools

import jax
import jax.numpy as jnp
from jax import lax
from jax.experimental import pallas as pl
from jax.experimental.pallas import tpu as pltpu
from jax.experimental.pallas import tpu_sc as plsc

N = 10000
NPAD = 10240
NC, NS, L = 2, 16, 16
NW = NC * NS
CHUNK = 128
NB = 8
LOOKAHEAD = 4
ROWS = 1000

_mesh = plsc.VectorSubcoreMesh(
    core_axis_name="c", subcore_axis_name="s", num_cores=NC, num_subcores=NS)

_sc_params = pltpu.CompilerParams(
    needs_layout_passes=False, use_tc_tiling_on_sc=False)


def _cdiv(a, b):
    return (a + b - 1) // b


def _make_deg(CH):
    @functools.partial(
        pl.kernel,
        out_type=jax.ShapeDtypeStruct((NW, NPAD), jnp.float32),
        mesh=_mesh,
        compiler_params=_sc_params,
        scratch_types=[
            pltpu.VMEM((CH, CHUNK), jnp.int32),
            pltpu.VMEM((NPAD,), jnp.float32),
        ],
    )
    def deg_kernel(col_hbm, out_hbm, colv, degv):
        c = lax.axis_index("c")
        s = lax.axis_index("s")
        w = c * NS + s
        pltpu.sync_copy(col_hbm.at[w], colv)

        def zbody(i, carry):
            degv[pl.ds(i * L, L)] = jnp.zeros((L,), jnp.float32)
            return carry

        lax.fori_loop(0, NPAD // L, zbody, 0)
        ones = jnp.ones((L,), jnp.float32)

        def body(j, carry):
            for k in range(CHUNK // L):
                idx = colv[j, pl.ds(k * L, L)]
                plsc.addupdate_scatter(degv, [idx], ones)
            return carry

        lax.fori_loop(0, CH, body, 0)
        pltpu.sync_copy(degv, out_hbm.at[w])

    return deg_kernel


def _make_agg(H, CH):
    rpt = NPAD // NS

    @functools.partial(
        pl.kernel,
        out_type=jax.ShapeDtypeStruct((NC, NPAD, H), jnp.float32),
        mesh=_mesh,
        compiler_params=_sc_params,
        scratch_types=(
            [pltpu.VMEM((CH, CHUNK), jnp.int32),
             pltpu.VMEM((CH, CHUNK), jnp.int32)]
            + [pltpu.VMEM((CHUNK, H), jnp.float32)] * NB
            + [pltpu.VMEM_SHARED((NPAD, H), jnp.float32)]
            + [pltpu.SemaphoreType.DMA] * (2 * NB)
        ),
    )
    def agg_kernel(g_hbm, row_hbm, col_hbm, out_hbm, rowv, colv, *rest):
        bufs = rest[:NB]
        zbuf = bufs[0]
        acc = rest[NB]
        gsem = rest[NB + 1:NB + 1 + NB]
        ssem = rest[NB + 1 + NB:NB + 1 + 2 * NB]

        c = lax.axis_index("c")
        s = lax.axis_index("s")
        w = c * NS + s
        pltpu.sync_copy(row_hbm.at[w], rowv)
        pltpu.sync_copy(col_hbm.at[w], colv)

        def zb(i, carry):
            for k in range(H // L):
                zbuf[i, pl.ds(k * L, L)] = jnp.zeros((L,), jnp.float32)
            return carry

        lax.fori_loop(0, CHUNK, zb, 0)
        for k in range(rpt // CHUNK):
            pltpu.sync_copy(zbuf, acc.at[pl.ds(s * rpt + k * CHUNK, CHUNK)])
        plsc.subcore_barrier()

        for p in range(LOOKAHEAD):
            pltpu.async_copy(g_hbm.at[rowv.at[p]], bufs[p], gsem[p])

        def body(t, carry):
            for p in range(NB):
                j = t * NB + p
                f = j + LOOKAHEAD
                pf = (p + LOOKAHEAD) % NB

                @pl.when(f < CH)
                def _(f=f, pf=pf):
                    @pl.when(f >= NB)
                    def _():
                        pltpu.make_async_copy(
                            bufs[pf], acc.at[colv.at[f - NB]], ssem[pf]
                        ).wait()
                    pltpu.async_copy(g_hbm.at[rowv.at[f]], bufs[pf], gsem[pf])

                pltpu.make_async_copy(
                    g_hbm.at[rowv.at[j]], bufs[p], gsem[p]).wait()
                pltpu.make_async_copy(
                    bufs[p], acc.at[colv.at[j]], ssem[p]).start(add=True)
            return carry

        lax.fori_loop(0, CH // NB, body, 0)
        for p in range(NB):
            pltpu.make_async_copy(
                bufs[p], acc.at[colv.at[CH - NB + p]], ssem[p]).wait()

        plsc.subcore_barrier()
        for k in range(rpt // CHUNK):
            r0 = s * rpt + k * CHUNK
            pltpu.sync_copy(acc.at[pl.ds(r0, CHUNK)], zbuf)
            pltpu.sync_copy(zbuf, out_hbm.at[c, pl.ds(r0, CHUNK)])

    return agg_kernel


def _scale_body(x_ref, w_ref, degp_ref, dinv_ref, g_ref):
    h1 = jnp.dot(x_ref[...], w_ref[...], preferred_element_type=jnp.float32)
    deg = jnp.sum(degp_ref[...], axis=1, keepdims=True) + 1.0
    dinv = lax.rsqrt(deg)
    dinv_ref[...] = dinv
    g_ref[...] = h1 * dinv


def _deg_scale(x, W1, degT):
    k = x.shape[1]
    h = W1.shape[1]
    return pl.pallas_call(
        _scale_body,
        grid=(N // ROWS,),
        in_specs=[pl.BlockSpec((ROWS, k), lambda i: (i, 0)),
                  pl.BlockSpec((k, h), lambda i: (0, 0)),
                  pl.BlockSpec((ROWS, NW), lambda i: (i, 0))],
        out_specs=[pl.BlockSpec((ROWS, 1), lambda i: (i, 0)),
                   pl.BlockSpec((ROWS, h), lambda i: (i, 0))],
        out_shape=[jax.ShapeDtypeStruct((N, 1), jnp.float32),
                   jax.ShapeDtypeStruct((N, h), jnp.float32)],
    )(x, W1, degT)


def _layer_body(p_ref, g_ref, dinv_ref, b_ref, w_ref, o_ref):
    agg = p_ref[0] + p_ref[1] + g_ref[...]
    outl = jnp.maximum(dinv_ref[...] * agg + b_ref[...], 0.0)
    o_ref[...] = dinv_ref[...] * jnp.dot(outl, w_ref[...],
                                         preferred_element_type=jnp.float32)


def _layer_step(p, g, dinv, b, Wn):
    h = g.shape[1]
    m = Wn.shape[1]
    return pl.pallas_call(
        _layer_body,
        grid=(N // ROWS,),
        in_specs=[pl.BlockSpec((NC, ROWS, h), lambda i: (0, i, 0)),
                  pl.BlockSpec((ROWS, h), lambda i: (i, 0)),
                  pl.BlockSpec((ROWS, 1), lambda i: (i, 0)),
                  pl.BlockSpec((1, h), lambda i: (0, 0)),
                  pl.BlockSpec((h, m), lambda i: (0, 0))],
        out_specs=pl.BlockSpec((ROWS, m), lambda i: (i, 0)),
        out_shape=jax.ShapeDtypeStruct((N, m), jnp.float32),
    )(p, g, dinv, b, Wn)


def _final_body(q_ref, g_ref, dinv_ref, b_ref, wl_ref, bl_ref, o_ref):
    agg = q_ref[0] + q_ref[1] + g_ref[...]
    outl = jnp.maximum(dinv_ref[...] * agg + b_ref[...], 0.0)
    logit = jnp.dot(outl, wl_ref[...],
                    preferred_element_type=jnp.float32) + bl_ref[...]
    o_ref[...] = jnp.concatenate([-logit, logit], axis=1)


def _final_step(q, g, dinv, b, Wl, bl):
    h = g.shape[1]
    return pl.pallas_call(
        _final_body,
        grid=(N // ROWS,),
        in_specs=[pl.BlockSpec((NC, ROWS, h), lambda i: (0, i, 0)),
                  pl.BlockSpec((ROWS, h), lambda i: (i, 0)),
                  pl.BlockSpec((ROWS, 1), lambda i: (i, 0)),
                  pl.BlockSpec((1, h), lambda i: (0, 0)),
                  pl.BlockSpec((h, 1), lambda i: (0, 0)),
                  pl.BlockSpec((1, 1), lambda i: (0, 0))],
        out_specs=pl.BlockSpec((ROWS, 2), lambda i: (i, 0)),
        out_shape=jax.ShapeDtypeStruct((N, 2), jnp.float32),
    )(q, g, dinv, b, Wl, bl)


def kernel(x, edge_index, W1, b1, W2, b2, Wl, bl):
    E = edge_index.shape[1]
    CH = _cdiv(E, NW * CHUNK)
    CH = _cdiv(CH, NB) * NB
    Epad = NW * CH * CHUNK
    pad = Epad - E
    pad_iota = jnp.arange(pad, dtype=jnp.int32)
    row = jnp.concatenate([edge_index[0], pad_iota % N])
    col = jnp.concatenate([edge_index[1], N + pad_iota % (NPAD - N)])
    row3 = row.reshape(NW, CH, CHUNK)
    col3 = col.reshape(NW, CH, CHUNK)

    degp = _make_deg(CH)(col3)
    dinv, g1 = _deg_scale(x, W1, degp[:, :N].T)

    p1 = _make_agg(g1.shape[1], CH)(g1, row3, col3)
    g2 = _layer_step(p1[:, :N, :], g1, dinv, b1.reshape(1, -1), W2)

    p2 = _make_agg(g2.shape[1], CH)(g2, row3, col3)
    return _final_step(p2[:, :N, :], g2, dinv, b2.reshape(1, -1),
                       Wl, bl.reshape(1, 1))


# speedup vs baseline: 48.7218x; 1.0707x over previous
import functools

import jax
import jax.numpy as jnp
from jax import lax
from jax.experimental import pallas as pl
from jax.experimental.pallas import tpu as pltpu
from jax.experimental.pallas import tpu_sc as plsc

N = 10000
NPAD = 10240
NC, NS, L = 2, 16, 16
NW = NC * NS
CHUNK = 128
NB = 8
LOOKAHEAD = 4
ROWS = 2000

_mesh = plsc.VectorSubcoreMesh(
    core_axis_name="c", subcore_axis_name="s", num_cores=NC, num_subcores=NS)

_sc_params = pltpu.CompilerParams(
    needs_layout_passes=False, use_tc_tiling_on_sc=False)


def _cdiv(a, b):
    return (a + b - 1) // b


def _make_deg(CH):
    @functools.partial(
        pl.kernel,
        out_type=jax.ShapeDtypeStruct((NW, NPAD), jnp.float32),
        mesh=_mesh,
        compiler_params=_sc_params,
        scratch_types=[
            pltpu.VMEM((CH, CHUNK), jnp.int32),
            pltpu.VMEM((NPAD,), jnp.float32),
        ],
    )
    def deg_kernel(col_hbm, out_hbm, colv, degv):
        c = lax.axis_index("c")
        s = lax.axis_index("s")
        w = c * NS + s
        pltpu.sync_copy(col_hbm.at[w], colv)

        def zbody(i, carry):
            degv[pl.ds(i * L, L)] = jnp.zeros((L,), jnp.float32)
            return carry

        lax.fori_loop(0, NPAD // L, zbody, 0)
        ones = jnp.ones((L,), jnp.float32)

        def body(j, carry):
            for k in range(CHUNK // L):
                idx = colv[j, pl.ds(k * L, L)]
                plsc.addupdate_scatter(degv, [idx], ones)
            return carry

        lax.fori_loop(0, CH, body, 0)
        pltpu.sync_copy(degv, out_hbm.at[w])

    return deg_kernel


def _make_agg(H, CH):
    rpt = NPAD // NS

    @functools.partial(
        pl.kernel,
        out_type=jax.ShapeDtypeStruct((NC, NPAD, H), jnp.float32),
        mesh=_mesh,
        compiler_params=_sc_params,
        scratch_types=(
            [pltpu.VMEM((CH, CHUNK), jnp.int32),
             pltpu.VMEM((CH, CHUNK), jnp.int32)]
            + [pltpu.VMEM((CHUNK, H), jnp.float32)] * NB
            + [pltpu.VMEM_SHARED((NPAD, H), jnp.float32)]
            + [pltpu.SemaphoreType.DMA] * (2 * NB)
        ),
    )
    def agg_kernel(g_hbm, row_hbm, col_hbm, out_hbm, rowv, colv, *rest):
        bufs = rest[:NB]
        zbuf = bufs[0]
        acc = rest[NB]
        gsem = rest[NB + 1:NB + 1 + NB]
        ssem = rest[NB + 1 + NB:NB + 1 + 2 * NB]

        c = lax.axis_index("c")
        s = lax.axis_index("s")
        w = c * NS + s
        pltpu.sync_copy(row_hbm.at[w], rowv)
        pltpu.sync_copy(col_hbm.at[w], colv)

        def zb(i, carry):
            for k in range(H // L):
                zbuf[i, pl.ds(k * L, L)] = jnp.zeros((L,), jnp.float32)
            return carry

        lax.fori_loop(0, CHUNK, zb, 0)
        for k in range(rpt // CHUNK):
            pltpu.sync_copy(zbuf, acc.at[pl.ds(s * rpt + k * CHUNK, CHUNK)])
        plsc.subcore_barrier()

        for p in range(LOOKAHEAD):
            pltpu.async_copy(g_hbm.at[rowv.at[p]], bufs[p], gsem[p])

        def body(t, carry):
            for p in range(NB):
                j = t * NB + p
                f = j + LOOKAHEAD
                pf = (p + LOOKAHEAD) % NB

                @pl.when(f < CH)
                def _(f=f, pf=pf):
                    @pl.when(f >= NB)
                    def _():
                        pltpu.make_async_copy(
                            bufs[pf], acc.at[colv.at[f - NB]], ssem[pf]
                        ).wait()
                    pltpu.async_copy(g_hbm.at[rowv.at[f]], bufs[pf], gsem[pf])

                pltpu.make_async_copy(
                    g_hbm.at[rowv.at[j]], bufs[p], gsem[p]).wait()
                pltpu.make_async_copy(
                    bufs[p], acc.at[colv.at[j]], ssem[p]).start(add=True)
            return carry

        lax.fori_loop(0, CH // NB, body, 0)
        for p in range(NB):
            pltpu.make_async_copy(
                bufs[p], acc.at[colv.at[CH - NB + p]], ssem[p]).wait()

        plsc.subcore_barrier()
        for k in range(rpt // CHUNK):
            r0 = s * rpt + k * CHUNK
            pltpu.sync_copy(acc.at[pl.ds(r0, CHUNK)], zbuf)
            pltpu.sync_copy(zbuf, out_hbm.at[c, pl.ds(r0, CHUNK)])

    return agg_kernel


def _scale_body(x_ref, w_ref, degp_ref, dinv_ref, g_ref):
    h1 = jnp.dot(x_ref[...], w_ref[...], preferred_element_type=jnp.float32)
    deg = jnp.sum(degp_ref[...], axis=1, keepdims=True) + 1.0
    dinv = lax.rsqrt(deg)
    dinv_ref[...] = dinv
    g_ref[...] = h1 * dinv


def _deg_scale(x, W1, degT):
    k = x.shape[1]
    h = W1.shape[1]
    return pl.pallas_call(
        _scale_body,
        grid=(N // ROWS,),
        in_specs=[pl.BlockSpec((ROWS, k), lambda i: (i, 0)),
                  pl.BlockSpec((k, h), lambda i: (0, 0)),
                  pl.BlockSpec((ROWS, NW), lambda i: (i, 0))],
        out_specs=[pl.BlockSpec((ROWS, 1), lambda i: (i, 0)),
                   pl.BlockSpec((ROWS, h), lambda i: (i, 0))],
        out_shape=[jax.ShapeDtypeStruct((N, 1), jnp.float32),
                   jax.ShapeDtypeStruct((N, h), jnp.float32)],
    )(x, W1, degT)


def _layer_body(p_ref, g_ref, dinv_ref, b_ref, w_ref, o_ref):
    agg = p_ref[0] + p_ref[1] + g_ref[...]
    outl = jnp.maximum(dinv_ref[...] * agg + b_ref[...], 0.0)
    o_ref[...] = dinv_ref[...] * jnp.dot(outl, w_ref[...],
                                         preferred_element_type=jnp.float32)


def _layer_step(p, g, dinv, b, Wn):
    h = g.shape[1]
    m = Wn.shape[1]
    return pl.pallas_call(
        _layer_body,
        grid=(N // ROWS,),
        in_specs=[pl.BlockSpec((NC, ROWS, h), lambda i: (0, i, 0)),

                  pl.BlockSpec((ROWS, h), lambda i: (i, 0)),
                  pl.BlockSpec((ROWS, 1), lambda i: (i, 0)),
                  pl.BlockSpec((1, h), lambda i: (0, 0)),
                  pl.BlockSpec((h, m), lambda i: (0, 0))],
        out_specs=pl.BlockSpec((ROWS, m), lambda i: (i, 0)),
        out_shape=jax.ShapeDtypeStruct((N, m), jnp.float32),
    )(p, g, dinv, b, Wn)


def _final_body(q_ref, g_ref, dinv_ref, b_ref, wl_ref, bl_ref, o_ref):
    agg = q_ref[0] + q_ref[1] + g_ref[...]
    outl = jnp.maximum(dinv_ref[...] * agg + b_ref[...], 0.0)
    o_ref[...] = jnp.dot(outl, wl_ref[...],
                         preferred_element_type=jnp.float32) + bl_ref[...]


def _final_step(q, g, dinv, b, Wl, bl):
    h = g.shape[1]
    return pl.pallas_call(
        _final_body,
        grid=(N // ROWS,),
        in_specs=[pl.BlockSpec((NC, ROWS, h), lambda i: (0, i, 0)),
                  pl.BlockSpec((ROWS, h), lambda i: (i, 0)),
                  pl.BlockSpec((ROWS, 1), lambda i: (i, 0)),
                  pl.BlockSpec((1, h), lambda i: (0, 0)),
                  pl.BlockSpec((h, 1), lambda i: (0, 0)),
                  pl.BlockSpec((1, 1), lambda i: (0, 0))],
        out_specs=pl.BlockSpec((ROWS, 1), lambda i: (i, 0)),
        out_shape=jax.ShapeDtypeStruct((N, 1), jnp.float32),
    )(q, g, dinv, b, Wl, bl)


def kernel(x, edge_index, W1, b1, W2, b2, Wl, bl):
    E = edge_index.shape[1]
    CH = _cdiv(E, NW * CHUNK)
    CH = _cdiv(CH, NB) * NB
    Epad = NW * CH * CHUNK
    pad = Epad - E
    pad_iota = jnp.arange(pad, dtype=jnp.int32)
    row = jnp.concatenate([edge_index[0], pad_iota % N])
    col = jnp.concatenate([edge_index[1], N + pad_iota % (NPAD - N)])
    row3 = row.reshape(NW, CH, CHUNK)
    col3 = col.reshape(NW, CH, CHUNK)

    degp = _make_deg(CH)(col3)
    dinv, g1 = _deg_scale(x, W1, degp[:, :N].T)

    p1 = _make_agg(g1.shape[1], CH)(g1, row3, col3)
    g2 = _layer_step(p1, g1, dinv, b1.reshape(1, -1), W2)

    p2 = _make_agg(g2.shape[1], CH)(g2, row3, col3)
    logit = _final_step(p2, g2, dinv, b2.reshape(1, -1),
                        Wl, bl.reshape(1, 1))[:, 0]
    return jnp.stack([-logit, logit], axis=1)


# speedup vs baseline: 55.1425x; 1.1318x over previous
import functools

import jax
import jax.numpy as jnp
from jax import lax
from jax.experimental import pallas as pl
from jax.experimental.pallas import tpu as pltpu
from jax.experimental.pallas import tpu_sc as plsc

N = 10000
NPAD = 10240
NC, NS, L = 2, 16, 16
NW = NC * NS
CHUNK = 128
NB = 8
LOOKAHEAD = 4
ROWS = 2000

_mesh = plsc.VectorSubcoreMesh(
    core_axis_name="c", subcore_axis_name="s", num_cores=NC, num_subcores=NS)

_sc_params = pltpu.CompilerParams(
    needs_layout_passes=False, use_tc_tiling_on_sc=False)


def _cdiv(a, b):
    return (a + b - 1) // b


def _make_deg(CH):
    @functools.partial(
        pl.kernel,
        out_type=jax.ShapeDtypeStruct((NW, NPAD), jnp.float32),
        mesh=_mesh,
        compiler_params=_sc_params,
        scratch_types=[
            pltpu.VMEM((CH, CHUNK), jnp.int32),
            pltpu.VMEM((NPAD,), jnp.float32),
        ],
    )
    def deg_kernel(col_hbm, out_hbm, colv, degv):
        c = lax.axis_index("c")
        s = lax.axis_index("s")
        w = c * NS + s
        pltpu.sync_copy(col_hbm.at[w], colv)

        def zbody(i, carry):
            degv[pl.ds(i * L, L)] = jnp.zeros((L,), jnp.float32)
            return carry

        lax.fori_loop(0, NPAD // L, zbody, 0)
        ones = jnp.ones((L,), jnp.float32)

        def body(j, carry):
            for k in range(CHUNK // L):
                idx = colv[j, pl.ds(k * L, L)]
                plsc.addupdate_scatter(degv, [idx], ones)
            return carry

        lax.fori_loop(0, CH, body, 0)
        pltpu.sync_copy(degv, out_hbm.at[w])

    return deg_kernel


def _make_agg(H, CH):
    rpt = NPAD // NS

    @functools.partial(
        pl.kernel,
        out_type=jax.ShapeDtypeStruct((NC, NPAD, 128), jnp.float32),
        mesh=_mesh,
        compiler_params=_sc_params,
        scratch_types=(
            [pltpu.VMEM((CH, CHUNK), jnp.int32),
             pltpu.VMEM((CH, CHUNK), jnp.int32)]
            + [pltpu.VMEM((CHUNK, H), jnp.float32)] * NB
            + [pltpu.VMEM_SHARED((NPAD, H), jnp.float32)]
            + [pltpu.SemaphoreType.DMA] * (2 * NB)
        ),
    )
    def agg_kernel(g_hbm, row_hbm, col_hbm, out_hbm, rowv, colv, *rest):
        bufs = rest[:NB]
        zbuf = bufs[0]
        acc = rest[NB]
        gsem = rest[NB + 1:NB + 1 + NB]
        ssem = rest[NB + 1 + NB:NB + 1 + 2 * NB]

        c = lax.axis_index("c")
        s = lax.axis_index("s")
        w = c * NS + s
        pltpu.sync_copy(row_hbm.at[w], rowv)
        pltpu.sync_copy(col_hbm.at[w], colv)

        def zb(i, carry):
            for k in range(H // L):
                zbuf[i, pl.ds(k * L, L)] = jnp.zeros((L,), jnp.float32)
            return carry

        lax.fori_loop(0, CHUNK, zb, 0)
        for k in range(rpt // CHUNK):
            pltpu.sync_copy(zbuf, acc.at[pl.ds(s * rpt + k * CHUNK, CHUNK)])
        plsc.subcore_barrier()

        for p in range(LOOKAHEAD):
            pltpu.async_copy(g_hbm.at[rowv.at[p]], bufs[p], gsem[p])

        def body(t, carry):
            for p in range(NB):
                j = t * NB + p
                f = j + LOOKAHEAD
                pf = (p + LOOKAHEAD) % NB

                @pl.when(f < CH)
                def _(f=f, pf=pf):
                    @pl.when(f >= NB)
                    def _():
                        pltpu.make_async_copy(
                            bufs[pf], acc.at[colv.at[f - NB]], ssem[pf]
                        ).wait()
                    pltpu.async_copy(g_hbm.at[rowv.at[f]], bufs[pf], gsem[pf])

                pltpu.make_async_copy(
                    g_hbm.at[rowv.at[j]], bufs[p], gsem[p]).wait()
                pltpu.make_async_copy(
                    bufs[p], acc.at[colv.at[j]], ssem[p]).start(add=True)
            return carry

        lax.fori_loop(0, CH // NB, body, 0)
        for p in range(NB):
            pltpu.make_async_copy(
                bufs[p], acc.at[colv.at[CH - NB + p]], ssem[p]).wait()

        plsc.subcore_barrier()
        for k in range(rpt // CHUNK):
            r0 = s * rpt + k * CHUNK
            pltpu.sync_copy(acc.at[pl.ds(r0, CHUNK)], zbuf)
            pltpu.sync_copy(zbuf, out_hbm.at[c, pl.ds(r0, CHUNK), pl.ds(0, H)])

    return agg_kernel


def _scale_body(x_ref, w_ref, degp_ref, dinv_ref, g_ref):
    h1 = jnp.dot(x_ref[...], w_ref[...], preferred_element_type=jnp.float32)
    deg = jnp.sum(degp_ref[...], axis=1, keepdims=True) + 1.0
    dinv = lax.rsqrt(deg)
    dinv_ref[...] = dinv
    g_ref[...] = h1 * dinv


def _deg_scale(x, W1, degT):
    k = x.shape[1]
    h = W1.shape[1]
    return pl.pallas_call(
        _scale_body,
        grid=(N // ROWS,),
        in_specs=[pl.BlockSpec((ROWS, k), lambda i: (i, 0)),
                  pl.BlockSpec((k, h), lambda i: (0, 0)),
                  pl.BlockSpec((ROWS, NW), lambda i: (i, 0))],
        out_specs=[pl.BlockSpec((ROWS, 1), lambda i: (i, 0)),
                   pl.BlockSpec((ROWS, h), lambda i: (i, 0))],
        out_shape=[jax.ShapeDtypeStruct((N, 1), jnp.float32),
                   jax.ShapeDtypeStruct((N, h), jnp.float32)],
    )(x, W1, degT)


def _layer_body(p_ref, g_ref, dinv_ref, b_ref, w_ref, o_ref):
    h = g_ref.shape[1]
    agg = p_ref[0, :, :h] + p_ref[1, :, :h] + g_ref[...]
    outl = jnp.maximum(dinv_ref[...] * agg + b_ref[...], 0.0)
    o_ref[...] = dinv_ref[...] * jnp.dot(outl, w_ref[...],
                                         preferred_element_type=jnp.float32)


def _layer_step(p, g, dinv, b, Wn):
    h = g.shape[1]
    m = Wn.shape[1]
    return pl.pallas_call(
        _layer_body,
        grid=(N // ROWS,),
        in_specs=[pl.BlockSpec((NC, ROWS, 128), lambda i: (0, i, 0)),
                  pl.BlockSpec((ROWS, h), lambda i: (i, 0)),
                  pl.BlockSpec((ROWS, 1), lambda i: (i, 0)),
                  pl.BlockSpec((1, h), lambda i: (0, 0)),
                  pl.BlockSpec((h, m), lambda i: (0, 0))],
        out_specs=pl.BlockSpec((ROWS, m), lambda i: (i, 0)),
        out_shape=jax.ShapeDtypeStruct((N, m), jnp.float32),
    )(p, g, dinv, b, Wn)


def _final_body(q_ref, g_ref, dinv_ref, b_ref, wl_ref, bl_ref, o_ref):
    h = g_ref.shape[1]
    agg = q_ref[0, :, :h] + q_ref[1, :, :h] + g_ref[...]
    outl = jnp.maximum(dinv_ref[...] * agg + b_ref[...], 0.0)
    logit = jnp.dot(outl, wl_ref[...],
                    preferred_element_type=jnp.float32) + bl_ref[...]
    o_ref[...] = jnp.concatenate([-logit, logit], axis=1)


def _final_step(q, g, dinv, b, Wl, bl):
    h = g.shape[1]
    return pl.pallas_call(
        _final_body,
        grid=(N // ROWS,),
        in_specs=[pl.BlockSpec((NC, ROWS, 128), lambda i: (0, i, 0)),
                  pl.BlockSpec((ROWS, h), lambda i: (i, 0)),
                  pl.BlockSpec((ROWS, 1), lambda i: (i, 0)),
                  pl.BlockSpec((1, h), lambda i: (0, 0)),
                  pl.BlockSpec((h, 1), lambda i: (0, 0)),
                  pl.BlockSpec((1, 1), lambda i: (0, 0))],
        out_specs=pl.BlockSpec((ROWS, 2), lambda i: (i, 0)),
        out_shape=jax.ShapeDtypeStruct((N, 2), jnp.float32),
    )(q, g, dinv, b, Wl, bl)


def kernel(x, edge_index, W1, b1, W2, b2, Wl, bl):
    E = edge_index.shape[1]
    CH = _cdiv(E, NW * CHUNK)
    CH = _cdiv(CH, NB) * NB
    Epad = NW * CH * CHUNK
    pad = Epad - E
    pad_iota = jnp.arange(pad, dtype=jnp.int32)
    row = jnp.concatenate([edge_index[0], pad_iota % N])
    col = jnp.concatenate([edge_index[1], N + pad_iota % (NPAD - N)])
    row3 = row.reshape(NW, CH, CHUNK)
    col3 = col.reshape(NW, CH, CHUNK)

    degp = _make_deg(CH)(col3)
    dinv, g1 = _deg_scale(x, W1, degp[:, :N].T)

    p1 = _make_agg(g1.shape[1], CH)(g1, row3, col3)
    g2 = _layer_step(p1, g1, dinv, b1.reshape(1, -1), W2)

    p2 = _make_agg(g2.shape[1], CH)(g2, row3, col3)
    return _final_step(p2, g2, dinv, b2.reshape(1, -1),
                       Wl, bl.reshape(1, 1))


# speedup vs baseline: 58.9308x; 1.0687x over previous
import functools

import jax
import jax.numpy as jnp
from jax import lax
from jax.experimental import pallas as pl
from jax.experimental.pallas import tpu as pltpu
from jax.experimental.pallas import tpu_sc as plsc

N = 10000
NPAD = 10240
NC, NS, L = 2, 16, 16
NW = NC * NS
CHUNK = 128
NB = 8
LOOKAHEAD = 4
ROWS = 2000

_mesh = plsc.VectorSubcoreMesh(
    core_axis_name="c", subcore_axis_name="s", num_cores=NC, num_subcores=NS)

_sc_params = pltpu.CompilerParams(
    needs_layout_passes=False, use_tc_tiling_on_sc=False)


def _cdiv(a, b):
    return (a + b - 1) // b


def _make_deg(CH):
    @functools.partial(
        pl.kernel,
        out_type=jax.ShapeDtypeStruct((NW, NPAD), jnp.float32),
        mesh=_mesh,
        compiler_params=_sc_params,
        scratch_types=[
            pltpu.VMEM((CH, CHUNK), jnp.int32),
            pltpu.VMEM((NPAD,), jnp.float32),
        ],
    )
    def deg_kernel(col_hbm, out_hbm, colv, degv):
        c = lax.axis_index("c")
        s = lax.axis_index("s")
        w = c * NS + s
        pltpu.sync_copy(col_hbm.at[w], colv)

        def zbody(i, carry):
            degv[pl.ds(i * L, L)] = jnp.zeros((L,), jnp.float32)
            return carry

        lax.fori_loop(0, NPAD // L, zbody, 0)
        ones = jnp.ones((L,), jnp.float32)

        def body(j, carry):
            for k in range(CHUNK // L):
                idx = colv[j, pl.ds(k * L, L)]
                plsc.addupdate_scatter(degv, [idx], ones)
            return carry

        lax.fori_loop(0, CH, body, 0)
        pltpu.sync_copy(degv, out_hbm.at[w])

    return deg_kernel


def _make_agg(H, CH):
    rpt = NPAD // NS

    @functools.partial(
        pl.kernel,
        out_type=jax.ShapeDtypeStruct((NC, NPAD, 128), jnp.float32),
        mesh=_mesh,
        compiler_params=_sc_params,
        scratch_types=(
            [pltpu.VMEM((CH, CHUNK), jnp.int32),
             pltpu.VMEM((CH, CHUNK), jnp.int32)]
            + [pltpu.VMEM((CHUNK, H), jnp.float32)] * NB
            + [pltpu.VMEM_SHARED((NPAD, H), jnp.float32)]
            + [pltpu.SemaphoreType.DMA] * (2 * NB)
        ),
    )
    def agg_kernel(g_hbm, row_hbm, col_hbm, out_hbm, rowv, colv, *rest):
        bufs = rest[:NB]
        zbuf = bufs[0]
        acc = rest[NB]
        gsem = rest[NB + 1:NB + 1 + NB]
        ssem = rest[NB + 1 + NB:NB + 1 + 2 * NB]

        c = lax.axis_index("c")
        s = lax.axis_index("s")
        w = c * NS + s
        pltpu.sync_copy(row_hbm.at[w], rowv)
        pltpu.sync_copy(col_hbm.at[w], colv)

        def zb(i, carry):
            for k in range(H // L):
                zbuf[i, pl.ds(k * L, L)] = jnp.zeros((L,), jnp.float32)
            return carry

        lax.fori_loop(0, CHUNK, zb, 0)
        for k in range(rpt // CHUNK):
            pltpu.sync_copy(zbuf, acc.at[pl.ds(s * rpt + k * CHUNK, CHUNK)])
        plsc.subcore_barrier()

        for p in range(LOOKAHEAD):
            pltpu.async_copy(g_hbm.at[rowv.at[p]], bufs[p], gsem[p])

        def body(t, carry):
            for p in range(NB):
                j = t * NB + p
                f = j + LOOKAHEAD
                pf = (p + LOOKAHEAD) % NB

                @pl.when(f < CH)
                def _(f=f, pf=pf):
                    @pl.when(f >= NB)
                    def _():
                        pltpu.make_async_copy(
                            bufs[pf], acc.at[colv.at[f - NB]], ssem[pf]
                        ).wait()
                    pltpu.async_copy(g_hbm.at[rowv.at[f]], bufs[pf], gsem[pf])

                pltpu.make_async_copy(
                    g_hbm.at[rowv.at[j]], bufs[p], gsem[p]).wait()
                pltpu.make_async_copy(
                    bufs[p], acc.at[colv.at[j]], ssem[p]).start(add=True)
            return carry

        lax.fori_loop(0, CH // NB, body, 0)
        for p in range(NB):
            pltpu.make_async_copy(
                bufs[p], acc.at[colv.at[CH - NB + p]], ssem[p]).wait()

        plsc.subcore_barrier()
        for k in range(rpt // CHUNK):
            r0 = s * rpt + k * CHUNK
            pltpu.sync_copy(acc.at[pl.ds(r0, CHUNK)], zbuf)
            pltpu.sync_copy(zbuf, out_hbm.at[c, pl.ds(r0, CHUNK), pl.ds(0, H)])

    return agg_kernel


def _scale_body(x_ref, w_ref, degp_ref, dinv_ref, g_ref):
    h1 = jnp.dot(x_ref[...], w_ref[...], preferred_element_type=jnp.float32)
    deg = jnp.sum(degp_ref[...], axis=1, keepdims=True) + 1.0
    dinv = lax.rsqrt(deg)
    dinv_ref[...] = dinv
    g_ref[...] = jnp.concatenate(
        [h1 * dinv, jnp.zeros((h1.shape[0], 128 - h1.shape[1]), jnp.float32)],
        axis=1)


def _deg_scale(x, W1, degT):
    k = x.shape[1]
    h = W1.shape[1]
    return pl.pallas_call(
        _scale_body,
        grid=(N // ROWS,),
        in_specs=[pl.BlockSpec((ROWS, k), lambda i: (i, 0)),
                  pl.BlockSpec((k, h), lambda i: (0, 0)),
                  pl.BlockSpec((ROWS, NW), lambda i: (i, 0))],
        out_specs=[pl.BlockSpec((ROWS, 1), lambda i: (i, 0)),
                   pl.BlockSpec((ROWS, 128), lambda i: (i, 0))],
        out_shape=[jax.ShapeDtypeStruct((N, 1), jnp.float32),
                   jax.ShapeDtypeStruct((N, 128), jnp.float32)],
    )(x, W1, degT)


def _layer_body(h, p_ref, g_ref, dinv_ref, b_ref, w_ref, o_ref):
    agg = p_ref[0, :, :h] + p_ref[1, :, :h] + g_ref[:, :h]
    outl = jnp.maximum(dinv_ref[...] * agg + b_ref[...], 0.0)
    g2 = dinv_ref[...] * jnp.dot(outl, w_ref[...],
                                 preferred_element_type=jnp.float32)
    o_ref[...] = jnp.concatenate(
        [g2, jnp.zeros((g2.shape[0], 128 - g2.shape[1]), jnp.float32)],
        axis=1)


def _layer_step(p, g, dinv, b, Wn, h):
    m = Wn.shape[1]
    return pl.pallas_call(
        functools.partial(_layer_body, h),
        grid=(N // ROWS,),
        in_specs=[pl.BlockSpec((NC, ROWS, 128), lambda i: (0, i, 0)),
                  pl.BlockSpec((ROWS, 128), lambda i: (i, 0)),
                  pl.BlockSpec((ROWS, 1), lambda i: (i, 0)),
                  pl.BlockSpec((1, h), lambda i: (0, 0)),
                  pl.BlockSpec((h, m), lambda i: (0, 0))],
        out_specs=pl.BlockSpec((ROWS, 128), lambda i: (i, 0)),
        out_shape=jax.ShapeDtypeStruct((N, 128), jnp.float32),
    )(p, g, dinv, b, Wn)


def _final_body(h, q_ref, g_ref, dinv_ref, b_ref, wl_ref, bl_ref, o_ref):
    agg = q_ref[0, :, :h] + q_ref[1, :, :h] + g_ref[:, :h]
    outl = jnp.maximum(dinv_ref[...] * agg + b_ref[...], 0.0)
    logit = jnp.dot(outl, wl_ref[...],
                    preferred_element_type=jnp.float32) + bl_ref[...]
    o_ref[...] = jnp.concatenate([-logit, logit], axis=1)


def _final_step(q, g, dinv, b, Wl, bl, h):
    return pl.pallas_call(
        functools.partial(_final_body, h),
        grid=(N // ROWS,),
        in_specs=[pl.BlockSpec((NC, ROWS, 128), lambda i: (0, i, 0)),
                  pl.BlockSpec((ROWS, 128), lambda i: (i, 0)),
                  pl.BlockSpec((ROWS, 1), lambda i: (i, 0)),
                  pl.BlockSpec((1, h), lambda i: (0, 0)),
                  pl.BlockSpec((h, 1), lambda i: (0, 0)),
                  pl.BlockSpec((1, 1), lambda i: (0, 0))],
        out_specs=pl.BlockSpec((ROWS, 2), lambda i: (i, 0)),
        out_shape=jax.ShapeDtypeStruct((N, 2), jnp.float32),
    )(q, g, dinv, b, Wl, bl)


def kernel(x, edge_index, W1, b1, W2, b2, Wl, bl):
    E = edge_index.shape[1]
    CH = _cdiv(E, NW * CHUNK)
    CH = _cdiv(CH, NB) * NB
    Epad = NW * CH * CHUNK
    pad = Epad - E
    pad_iota = jnp.arange(pad, dtype=jnp.int32)
    row = jnp.concatenate([edge_index[0], pad_iota % N])
    col = jnp.concatenate([edge_index[1], N + pad_iota % (NPAD - N)])
    row3a = (row * 2).reshape(NW, CH, CHUNK)
    row3b = (row * 4).reshape(NW, CH, CHUNK)
    col3 = col.reshape(NW, CH, CHUNK)

    degp = _make_deg(CH)(col3)
    dinv, g1 = _deg_scale(x, W1, degp[:, :N].T)

    p1 = _make_agg(W1.shape[1], CH)(
        g1.reshape(2 * N, 64), row3a, col3)
    g2 = _layer_step(p1, g1, dinv, b1.reshape(1, -1), W2, W1.shape[1])

    p2 = _make_agg(W2.shape[1], CH)(
        g2.reshape(4 * N, 32), row3b, col3)
    return _final_step(p2, g2, dinv, b2.reshape(1, -1),
                       Wl, bl.reshape(1, 1), W2.shape[1])


# speedup vs baseline: 59.3210x; 1.0066x over previous
import functools

import jax
import jax.numpy as jnp
from jax import lax
from jax.experimental import pallas as pl
from jax.experimental.pallas import tpu as pltpu
from jax.experimental.pallas import tpu_sc as plsc

N = 10000
NPAD = 10240
NC, NS, L = 2, 16, 16
NW = NC * NS
CHUNK = 128
NB = 8
LOOKAHEAD = 4
ROWS = 2000

_mesh = plsc.VectorSubcoreMesh(
    core_axis_name="c", subcore_axis_name="s", num_cores=NC, num_subcores=NS)

_sc_params = pltpu.CompilerParams(
    needs_layout_passes=False, use_tc_tiling_on_sc=False)


def _cdiv(a, b):
    return (a + b - 1) // b


def _make_deg(CH, NQ):
    full, extra = NQ // NW, NQ % NW

    @functools.partial(
        pl.kernel,
        out_type=[jax.ShapeDtypeStruct((NW, NPAD), jnp.float32),
                  jax.ShapeDtypeStruct((NW, CH, CHUNK), jnp.int32),
                  jax.ShapeDtypeStruct((NW, CH, CHUNK), jnp.int32),
                  jax.ShapeDtypeStruct((NW, CH, CHUNK), jnp.int32)],
        mesh=_mesh,
        compiler_params=_sc_params,
        scratch_types=[
            pltpu.VMEM((full + 1, 2 * CHUNK), jnp.int32),
            pltpu.VMEM((NPAD,), jnp.float32),
            pltpu.VMEM((CH, CHUNK), jnp.int32),
            pltpu.VMEM((CH, CHUNK), jnp.int32),
            pltpu.VMEM((CH, CHUNK), jnp.int32),
        ],
    )
    def deg_kernel(e_hbm, deg_hbm, r2_hbm, r4_hbm, col_hbm,
                   ev, degv, r2v, r4v, colv):
        c = lax.axis_index("c")
        s = lax.axis_index("s")
        w = c * NS + s
        base = w * full + jnp.minimum(w, extra)
        cnt = full + jnp.where(w < extra, 1, 0)
        pltpu.sync_copy(e_hbm.at[pl.ds(base, full)], ev.at[pl.ds(0, full)])

        @pl.when(w < extra)
        def _():
            pltpu.sync_copy(e_hbm.at[pl.ds(base + full, 1)],
                            ev.at[pl.ds(full, 1)])

        def zbody(i, carry):
            degv[pl.ds(i * L, L)] = jnp.zeros((L,), jnp.float32)
            return carry

        lax.fori_loop(0, NPAD // L, zbody, 0)
        ones = jnp.ones((L,), jnp.float32)

        def body(j, carry):
            for k in range(CHUNK // L):
                r = ev[j, pl.ds(k * L, L)]
                cc = ev[j, pl.ds(CHUNK + k * L, L)]
                plsc.addupdate_scatter(degv, [cc], ones)
                r2v[j, pl.ds(k * L, L)] = r * 2
                r4v[j, pl.ds(k * L, L)] = r * 4
                colv[j, pl.ds(k * L, L)] = cc
            return carry

        lax.fori_loop(0, cnt, body, 0)

        def pbody(j, carry):
            for k in range(CHUNK // L):
                u = lax.iota(jnp.int32, L) + (j * CHUNK + k * L + w * 331)
                pr = lax.rem(u, N)
                r2v[j, pl.ds(k * L, L)] = pr * 2
                r4v[j, pl.ds(k * L, L)] = pr * 4
                colv[j, pl.ds(k * L, L)] = N + lax.rem(u, NPAD - N)
            return carry

        lax.fori_loop(cnt, CH, pbody, 0)
        pltpu.sync_copy(degv, deg_hbm.at[w])
        pltpu.sync_copy(r2v, r2_hbm.at[w])
        pltpu.sync_copy(r4v, r4_hbm.at[w])
        pltpu.sync_copy(colv, col_hbm.at[w])

    return deg_kernel


def _make_agg(H, CH):
    rpt = NPAD // NS

    @functools.partial(
        pl.kernel,
        out_type=jax.ShapeDtypeStruct((NC, NPAD, 128), jnp.float32),
        mesh=_mesh,
        compiler_params=_sc_params,
        scratch_types=(
            [pltpu.VMEM((CH, CHUNK), jnp.int32),
             pltpu.VMEM((CH, CHUNK), jnp.int32)]
            + [pltpu.VMEM((CHUNK, H), jnp.float32)] * NB
            + [pltpu.VMEM_SHARED((NPAD, H), jnp.float32)]
            + [pltpu.SemaphoreType.DMA] * (2 * NB)
        ),
    )
    def agg_kernel(g_hbm, row_hbm, col_hbm, out_hbm, rowv, colv, *rest):
        bufs = rest[:NB]
        zbuf = bufs[0]
        acc = rest[NB]
        gsem = rest[NB + 1:NB + 1 + NB]
        ssem = rest[NB + 1 + NB:NB + 1 + 2 * NB]

        c = lax.axis_index("c")
        s = lax.axis_index("s")
        w = c * NS + s
        pltpu.sync_copy(row_hbm.at[w], rowv)
        pltpu.sync_copy(col_hbm.at[w], colv)

        def zb(i, carry):
            for k in range(H // L):
                zbuf[i, pl.ds(k * L, L)] = jnp.zeros((L,), jnp.float32)
            return carry

        lax.fori_loop(0, CHUNK, zb, 0)
        for k in range(rpt // CHUNK):
            pltpu.sync_copy(zbuf, acc.at[pl.ds(s * rpt + k * CHUNK, CHUNK)])
        plsc.subcore_barrier()

        for p in range(LOOKAHEAD):
            pltpu.async_copy(g_hbm.at[rowv.at[p]], bufs[p], gsem[p])

        def body(t, carry):
            for p in range(NB):
                j = t * NB + p
                f = j + LOOKAHEAD
                pf = (p + LOOKAHEAD) % NB

                @pl.when(f < CH)
                def _(f=f, pf=pf):
                    @pl.when(f >= NB)
                    def _():
                        pltpu.make_async_copy(
                            bufs[pf], acc.at[colv.at[f - NB]], ssem[pf]
                        ).wait()
                    pltpu.async_copy(g_hbm.at[rowv.at[f]], bufs[pf], gsem[pf])

                pltpu.make_async_copy(
                    g_hbm.at[rowv.at[j]], bufs[p], gsem[p]).wait()
                pltpu.make_async_copy(
                    bufs[p], acc.at[colv.at[j]], ssem[p]).start(add=True)
            return carry

        lax.fori_loop(0, CH // NB, body, 0)
        for p in range(NB):
            pltpu.make_async_copy(
                bufs[p], acc.at[colv.at[CH - NB + p]], ssem[p]).wait()

        plsc.subcore_barrier()
        for k in range(rpt // CHUNK):
            r0 = s * rpt + k * CHUNK
            pltpu.sync_copy(acc.at[pl.ds(r0, CHUNK)], zbuf)
            pltpu.sync_copy(zbuf, out_hbm.at[c, pl.ds(r0, CHUNK), pl.ds(0, H)])

    return agg_kernel


def _scale_body(x_ref, w_ref, degp_ref, dinv_ref, g_ref):
    h1 = jnp.dot(x_ref[...], w_ref[...], preferred_element_type=jnp.float32)
    deg = jnp.sum(degp_ref[...], axis=1, keepdims=True) + 1.0
    dinv = lax.rsqrt(deg)
    dinv_ref[...] = dinv
    g_ref[...] = jnp.concatenate(
        [h1 * dinv, jnp.zeros((h1.shape[0], 128 - h1.shape[1]), jnp.float32)],
        axis=1)


def _deg_scale(x, W1, degT):
    k = x.shape[1]
    h = W1.shape[1]
    return pl.pallas_call(
        _scale_body,
        grid=(N // ROWS,),
        in_specs=[pl.BlockSpec((ROWS, k), lambda i: (i, 0)),
                  pl.BlockSpec((k, h), lambda i: (0, 0)),
                  pl.BlockSpec((ROWS, NW), lambda i: (i, 0))],
        out_specs=[pl.BlockSpec((ROWS, 1), lambda i: (i, 0)),
                   pl.BlockSpec((ROWS, 128), lambda i: (i, 0))],
        out_shape=[jax.ShapeDtypeStruct((N, 1), jnp.float32),
                   jax.ShapeDtypeStruct((N, 128), jnp.float32)],
    )(x, W1, degT)


def _layer_body(h, p_ref, g_ref, dinv_ref, b_ref, w_ref, o_ref):
    agg = p_ref[0, :, :h] + p_ref[1, :, :h] + g_ref[:, :h]
    outl = jnp.maximum(dinv_ref[...] * agg + b_ref[...], 0.0)
    g2 = dinv_ref[...] * jnp.dot(outl, w_ref[...],
                                 preferred_element_type=jnp.float32)
    o_ref[...] = jnp.concatenate(
        [g2, jnp.zeros((g2.shape[0], 128 - g2.shape[1]), jnp.float32)],
        axis=1)


def _layer_step(p, g, dinv, b, Wn, h):
    m = Wn.shape[1]
    return pl.pallas_call(
        functools.partial(_layer_body, h),
        grid=(N // ROWS,),
        in_specs=[pl.BlockSpec((NC, ROWS, 128), lambda i: (0, i, 0)),
                  pl.BlockSpec((ROWS, 128), lambda i: (i, 0)),
                  pl.BlockSpec((ROWS, 1), lambda i: (i, 0)),
                  pl.BlockSpec((1, h), lambda i: (0, 0)),
                  pl.BlockSpec((h, m), lambda i: (0, 0))],
        out_specs=pl.BlockSpec((ROWS, 128), lambda i: (i, 0)),
        out_shape=jax.ShapeDtypeStruct((N, 128), jnp.float32),
    )(p, g, dinv, b, Wn)


def _final_body(h, q_ref, g_ref, dinv_ref, b_ref, wl_ref, bl_ref, o_ref):
    agg = q_ref[0, :, :h] + q_ref[1, :, :h] + g_ref[:, :h]
    outl = jnp.maximum(dinv_ref[...] * agg + b_ref[...], 0.0)
    logit = jnp.dot(outl, wl_ref[...],
                    preferred_element_type=jnp.float32) + bl_ref[...]
    o_ref[...] = jnp.concatenate([-logit, logit], axis=1)


def _final_step(q, g, dinv, b, Wl, bl, h):
    return pl.pallas_call(
        functools.partial(_final_body, h),
        grid=(N // ROWS,),
        in_specs=[pl.BlockSpec((NC, ROWS, 128), lambda i: (0, i, 0)),
                  pl.BlockSpec((ROWS, 128), lambda i: (i, 0)),
                  pl.BlockSpec((ROWS, 1), lambda i: (i, 0)),
                  pl.BlockSpec((1, h), lambda i: (0, 0)),
                  pl.BlockSpec((h, 1), lambda i: (0, 0)),
                  pl.BlockSpec((1, 1), lambda i: (0, 0))],
        out_specs=pl.BlockSpec((ROWS, 2), lambda i: (i, 0)),
        out_shape=jax.ShapeDtypeStruct((N, 2), jnp.float32),
    )(q, g, dinv, b, Wl, bl)


def kernel(x, edge_index, W1, b1, W2, b2, Wl, bl):
    E = edge_index.shape[1]
    NQ = E // CHUNK
    CH = _cdiv(_cdiv(NQ, NW), NB) * NB
    et = jnp.transpose(edge_index.reshape(2, NQ, CHUNK),
                       (1, 0, 2)).reshape(NQ, 2 * CHUNK)

    degp, row3a, row3b, col3 = _make_deg(CH, NQ)(et)
    dinv, g1 = _deg_scale(x, W1, degp[:, :N].T)

    p1 = _make_agg(W1.shape[1], CH)(
        g1.reshape(2 * N, 64), row3a, col3)
    g2 = _layer_step(p1, g1, dinv, b1.reshape(1, -1), W2, W1.shape[1])

    p2 = _make_agg(W2.shape[1], CH)(
        g2.reshape(4 * N, 32), row3b, col3)
    return _final_step(p2, g2, dinv, b2.reshape(1, -1),
                       Wl, bl.reshape(1, 1), W2.shape[1])
